# baseline (speedup 1.0000x reference)
import functools

import jax
import jax.numpy as jnp
from jax import lax
from jax.experimental import pallas as pl
from jax.experimental.pallas import tpu as pltpu
from jax.experimental.pallas import tpu_sc as plsc

N_NODES = 10000
N_EDGES = 320000
D = 128
DH = D // 2

NS = 16
EPS = N_EDGES // NS
C = 50
CHUNKS = EPS // C
NBUF = 10
ZR = 80
RPT = 640
RPT_LAST = N_NODES - 15 * RPT


def _make_sc_pass(with_deg: bool):
    mesh = plsc.VectorSubcoreMesh(core_axis_name="c", subcore_axis_name="s")
    outs = [jax.ShapeDtypeStruct((N_NODES, D), jnp.float32)]
    if with_deg:
        outs.append(jax.ShapeDtypeStruct((2, N_NODES), jnp.float32))
    scratch = [
        pltpu.VMEM((CHUNKS, C), jnp.int32),
        pltpu.VMEM((CHUNKS, C), jnp.int32),
        [pltpu.VMEM((C, DH), jnp.float32)] * NBUF,
        [pltpu.SemaphoreType.DMA] * NBUF,
        [pltpu.SemaphoreType.DMA] * NBUF,
        pltpu.SemaphoreType.DMA,
        pltpu.SemaphoreType.DMA,
        pltpu.VMEM((ZR, DH), jnp.float32),
        pltpu.VMEM_SHARED((N_NODES, DH), jnp.float32),
    ]
    if with_deg:
        scratch += [
            pltpu.VMEM((2000,), jnp.float32),
            pltpu.VMEM((128,), jnp.float32),
            pltpu.SemaphoreType.DMA,
            pltpu.VMEM_SHARED((N_NODES,), jnp.float32),
        ]

    out_type = tuple(outs) if with_deg else outs[0]

    @functools.partial(pl.kernel, out_type=out_type, mesh=mesh,
                       scratch_types=scratch,
                       compiler_params=pltpu.CompilerParams(
                           use_tc_tiling_on_sc=False))
    def sc_pass(h_hbm, src_hbm, dst_hbm, pagg_hbm, *rest):
        if with_deg:
            (pdeg_hbm, src_l, dst_l, bufs, gsems, ssems, isem, zsem, zbuf,
             agg_sh, zdeg, ones_v, dsem, deg_sh) = rest
        else:
            src_l, dst_l, bufs, gsems, ssems, isem, zsem, zbuf, agg_sh = rest
        c = lax.axis_index("c")
        s = lax.axis_index("s")

        pltpu.async_copy(src_hbm.at[c, s], src_l, isem)
        pltpu.async_copy(dst_hbm.at[s], dst_l, isem)

        zero16 = jnp.zeros((16,), jnp.float32)

        @pl.loop(0, ZR)
        def _(i):
            @pl.loop(0, DH // 16)
            def _(j):
                zbuf[i, pl.ds(j * 16, 16)] = zero16

        @pl.when(s < 15)
        def _():
            @pl.loop(0, RPT // ZR)
            def _(t):
                pltpu.async_copy(zbuf, agg_sh.at[pl.ds(s * RPT + t * ZR, ZR)],
                                 zsem)

        @pl.when(s == 15)
        def _():
            @pl.loop(0, RPT_LAST // ZR)
            def _(t):
                pltpu.async_copy(zbuf,
                                 agg_sh.at[pl.ds(15 * RPT + t * ZR, ZR)], zsem)

        if with_deg:
            @pl.loop(0, 2000 // 16)
            def _(i):
                zdeg[pl.ds(i * 16, 16)] = zero16

            @pl.loop(0, 128 // 16)
            def _(i):
                ones_v[pl.ds(i * 16, 16)] = jnp.ones((16,), jnp.float32)

            @pl.when(s == 0)
            def _():
                @pl.loop(0, N_NODES // 2000)
                def _(t):
                    pltpu.async_copy(zdeg, deg_sh.at[pl.ds(t * 2000, 2000)],
                                     zsem)

        pltpu.make_async_copy(src_hbm.at[c, s], src_l, isem).wait()
        pltpu.make_async_copy(dst_hbm.at[s], dst_l, isem).wait()

        @pl.when(s < 15)
        def _():
            @pl.loop(0, RPT // ZR)
            def _(t):
                pltpu.make_async_copy(
                    zbuf, agg_sh.at[pl.ds(s * RPT + t * ZR, ZR)], zsem).wait()

        @pl.when(s == 15)
        def _():
            @pl.loop(0, RPT_LAST // ZR)
            def _(t):
                pltpu.make_async_copy(
                    zbuf, agg_sh.at[pl.ds(15 * RPT + t * ZR, ZR)],
                    zsem).wait()

        if with_deg:
            @pl.when(s == 0)
            def _():
                @pl.loop(0, N_NODES // 2000)
                def _(t):
                    pltpu.make_async_copy(
                        zdeg, deg_sh.at[pl.ds(t * 2000, 2000)], zsem).wait()

        plsc.subcore_barrier()

        def gather_src(k):
            return h_hbm.at[src_l.at[k]]

        for i in range(NBUF):
            pltpu.async_copy(gather_src(i), bufs[i], gsems[i])

        @pl.loop(0, CHUNKS, step=NBUF)
        def _(k):
            for i in range(NBUF):
                kk = k + i
                pltpu.make_async_copy(gather_src(kk), bufs[i],
                                      gsems[i]).wait()
                pltpu.async_copy(bufs[i], agg_sh.at[dst_l.at[kk]], ssems[i],
                                 add=True)
                if with_deg:
                    @pl.when((kk < CHUNKS // 2) == (c == 0))
                    def _():
                        pltpu.async_copy(ones_v.at[pl.ds(0, C)],
                                         deg_sh.at[dst_l.at[kk]], dsem,
                                         add=True)
            for i in range(NBUF):
                kk = k + i
                pltpu.make_async_copy(bufs[i], agg_sh.at[dst_l.at[kk]],
                                      ssems[i]).wait()

                @pl.when(kk + NBUF < CHUNKS)
                def _():
                    pltpu.async_copy(gather_src(kk + NBUF), bufs[i], gsems[i])

        if with_deg:
            @pl.loop(0, CHUNKS // 2)
            def _(k):
                pltpu.make_async_copy(ones_v.at[pl.ds(0, C)],
                                      deg_sh.at[dst_l.at[0]], dsem).wait()

        plsc.subcore_barrier()

        col0 = c * DH

        @pl.when(s < 15)
        def _():
            pltpu.sync_copy(agg_sh.at[pl.ds(s * RPT, RPT)],
                            pagg_hbm.at[pl.ds(s * RPT, RPT),
                                        pl.ds(col0, DH)])

        @pl.when(s == 15)
        def _():
            pltpu.sync_copy(agg_sh.at[pl.ds(15 * RPT, RPT_LAST)],
                            pagg_hbm.at[pl.ds(15 * RPT, RPT_LAST),
                                        pl.ds(col0, DH)])

        if with_deg:
            @pl.when(s == 0)
            def _():
                pltpu.sync_copy(deg_sh, pdeg_hbm.at[c])

    return sc_pass


_sc_pass_deg = _make_sc_pass(True)
_sc_pass = _make_sc_pass(False)

BLK = 5000


def _tc_self_body(x_ref, w_ref, b_ref, o_ref):
    o_ref[...] = jnp.dot(x_ref[...], w_ref[...],
                         preferred_element_type=jnp.float32) + b_ref[...]


def _tc_self(h, w, b):
    return pl.pallas_call(
        _tc_self_body,
        grid=(N_NODES // BLK,),
        in_specs=[
            pl.BlockSpec((BLK, D), lambda i: (i, 0)),
            pl.BlockSpec((D, D), lambda i: (0, 0)),
            pl.BlockSpec((1, D), lambda i: (0, 0)),
        ],
        out_specs=pl.BlockSpec((BLK, D), lambda i: (i, 0)),
        out_shape=jax.ShapeDtypeStruct((N_NODES, D), jnp.float32),
    )(h, w[:D], b)


def _tc_combine_body(relu, self_ref, pagg_ref, pdeg_ref, w_ref, o_ref):
    deg = jnp.maximum(jnp.sum(pdeg_ref[...], axis=1), 1.0)
    agg = pagg_ref[...] * (1.0 / deg)[:, None]
    acc = self_ref[...] + jnp.dot(agg, w_ref[...],
                                  preferred_element_type=jnp.float32)
    if relu:
        acc = jnp.maximum(acc, 0.0)
    o_ref[...] = acc


def _tc_combine(relu, self_, pagg, pdeg, w):
    return pl.pallas_call(
        functools.partial(_tc_combine_body, relu),
        grid=(N_NODES // BLK,),
        in_specs=[
            pl.BlockSpec((BLK, D), lambda i: (i, 0)),
            pl.BlockSpec((BLK, D), lambda i: (i, 0)),
            pl.BlockSpec((BLK, 2), lambda i: (i, 0)),
            pl.BlockSpec((D, D), lambda i: (0, 0)),
        ],
        out_specs=pl.BlockSpec((BLK, D), lambda i: (i, 0)),
        out_shape=jax.ShapeDtypeStruct((N_NODES, D), jnp.float32),
    )(self_, pagg, pdeg, w[D:])


def kernel(x, edge_index, W1, b1, W2, b2):
    src = edge_index[0]
    src01 = jnp.stack([2 * src, 2 * src + 1]).reshape(2, NS, CHUNKS, C)
    dst3 = edge_index[1].reshape(NS, CHUNKS, C)
    self1 = _tc_self(x, W1, b1.reshape(1, D))
    pagg1, pdeg = _sc_pass_deg(x.reshape(2 * N_NODES, DH), src01, dst3)
    pdeg2d = pdeg.T
    h1 = _tc_combine(True, self1, pagg1, pdeg2d, W1)
    self2 = _tc_self(h1, W2, b2.reshape(1, D))
    pagg2 = _sc_pass(h1.reshape(2 * N_NODES, DH), src01, dst3)
    h2 = _tc_combine(False, self2, pagg2, pdeg2d, W2)
    return h2


# speedup vs baseline: 1.1455x; 1.1455x over previous
import functools

import jax
import jax.numpy as jnp
from jax import lax
from jax.experimental import pallas as pl
from jax.experimental.pallas import tpu as pltpu
from jax.experimental.pallas import tpu_sc as plsc

N_NODES = 10000
N_EDGES = 320000
D = 128
DH = D // 2

NS = 16
EPS = N_EDGES // NS
C = 125
CHUNKS = EPS // C
NBUF = 5
ZR = 80
RPT = 640
RPT_LAST = N_NODES - 15 * RPT


def _make_sc_pass(with_deg: bool):
    mesh = plsc.VectorSubcoreMesh(core_axis_name="c", subcore_axis_name="s")
    outs = [jax.ShapeDtypeStruct((N_NODES, D), jnp.float32)]
    if with_deg:
        outs.append(jax.ShapeDtypeStruct((2, N_NODES), jnp.float32))
    scratch = [
        pltpu.VMEM((CHUNKS, C), jnp.int32),
        pltpu.VMEM((CHUNKS, C), jnp.int32),
        [pltpu.VMEM((C, DH), jnp.float32)] * NBUF,
        [pltpu.SemaphoreType.DMA] * NBUF,
        [pltpu.SemaphoreType.DMA] * NBUF,
        pltpu.SemaphoreType.DMA,
        pltpu.SemaphoreType.DMA,
        pltpu.VMEM((ZR, DH), jnp.float32),
        pltpu.VMEM_SHARED((N_NODES, DH), jnp.float32),
    ]
    if with_deg:
        scratch += [
            pltpu.VMEM((2000,), jnp.float32),
            pltpu.VMEM((128,), jnp.float32),
            pltpu.SemaphoreType.DMA,
            pltpu.VMEM_SHARED((N_NODES,), jnp.float32),
        ]

    out_type = tuple(outs) if with_deg else outs[0]

    @functools.partial(pl.kernel, out_type=out_type, mesh=mesh,
                       scratch_types=scratch,
                       compiler_params=pltpu.CompilerParams(
                           use_tc_tiling_on_sc=False))
    def sc_pass(h_hbm, src_hbm, dst_hbm, pagg_hbm, *rest):
        if with_deg:
            (pdeg_hbm, src_l, dst_l, bufs, gsems, ssems, isem, zsem, zbuf,
             agg_sh, zdeg, ones_v, dsem, deg_sh) = rest
        else:
            src_l, dst_l, bufs, gsems, ssems, isem, zsem, zbuf, agg_sh = rest
        c = lax.axis_index("c")
        s = lax.axis_index("s")

        pltpu.async_copy(src_hbm.at[c, s], src_l, isem)
        pltpu.async_copy(dst_hbm.at[s], dst_l, isem)

        zero16 = jnp.zeros((16,), jnp.float32)

        @pl.loop(0, ZR)
        def _(i):
            @pl.loop(0, DH // 16)
            def _(j):
                zbuf[i, pl.ds(j * 16, 16)] = zero16

        @pl.when(s < 15)
        def _():
            @pl.loop(0, RPT // ZR)
            def _(t):
                pltpu.async_copy(zbuf, agg_sh.at[pl.ds(s * RPT + t * ZR, ZR)],
                                 zsem)

        @pl.when(s == 15)
        def _():
            @pl.loop(0, RPT_LAST // ZR)
            def _(t):
                pltpu.async_copy(zbuf,
                                 agg_sh.at[pl.ds(15 * RPT + t * ZR, ZR)], zsem)

        if with_deg:
            @pl.loop(0, 2000 // 16)
            def _(i):
                zdeg[pl.ds(i * 16, 16)] = zero16

            @pl.loop(0, 128 // 16)
            def _(i):
                ones_v[pl.ds(i * 16, 16)] = jnp.ones((16,), jnp.float32)

            @pl.when(s == 0)
            def _():
                @pl.loop(0, N_NODES // 2000)
                def _(t):
                    pltpu.async_copy(zdeg, deg_sh.at[pl.ds(t * 2000, 2000)],
                                     zsem)

        pltpu.make_async_copy(src_hbm.at[c, s], src_l, isem).wait()
        pltpu.make_async_copy(dst_hbm.at[s], dst_l, isem).wait()

        @pl.when(s < 15)
        def _():
            @pl.loop(0, RPT // ZR)
            def _(t):
                pltpu.make_async_copy(
                    zbuf, agg_sh.at[pl.ds(s * RPT + t * ZR, ZR)], zsem).wait()

        @pl.when(s == 15)
        def _():
            @pl.loop(0, RPT_LAST // ZR)
            def _(t):
                pltpu.make_async_copy(
                    zbuf, agg_sh.at[pl.ds(15 * RPT + t * ZR, ZR)],
                    zsem).wait()

        if with_deg:
            @pl.when(s == 0)
            def _():
                @pl.loop(0, N_NODES // 2000)
                def _(t):
                    pltpu.make_async_copy(
                        zdeg, deg_sh.at[pl.ds(t * 2000, 2000)], zsem).wait()

        plsc.subcore_barrier()

        def gather_src(k):
            return h_hbm.at[src_l.at[k]]

        for i in range(NBUF):
            pltpu.async_copy(gather_src(i), bufs[i], gsems[i])

        @pl.loop(0, CHUNKS, step=NBUF)
        def _(k):
            for i in range(NBUF):
                kk = k + i
                pltpu.make_async_copy(gather_src(kk), bufs[i],
                                      gsems[i]).wait()
                pltpu.async_copy(bufs[i], agg_sh.at[dst_l.at[kk]], ssems[i],
                                 add=True)
                if with_deg:
                    @pl.when((kk < CHUNKS // 2) == (c == 0))
                    def _():
                        pltpu.async_copy(ones_v.at[pl.ds(0, C)],
                                         deg_sh.at[dst_l.at[kk]], dsem,
                                         add=True)
            for i in range(NBUF):
                kk = k + i
                pltpu.make_async_copy(bufs[i], agg_sh.at[dst_l.at[kk]],
                                      ssems[i]).wait()

                @pl.when(kk + NBUF < CHUNKS)
                def _():
                    pltpu.async_copy(gather_src(kk + NBUF), bufs[i], gsems[i])

        if with_deg:
            @pl.loop(0, CHUNKS // 2)
            def _(k):
                pltpu.make_async_copy(ones_v.at[pl.ds(0, C)],
                                      deg_sh.at[dst_l.at[0]], dsem).wait()

        plsc.subcore_barrier()

        col0 = c * DH

        @pl.when(s < 15)
        def _():
            pltpu.sync_copy(agg_sh.at[pl.ds(s * RPT, RPT)],
                            pagg_hbm.at[pl.ds(s * RPT, RPT),
                                        pl.ds(col0, DH)])

        @pl.when(s == 15)
        def _():
            pltpu.sync_copy(agg_sh.at[pl.ds(15 * RPT, RPT_LAST)],
                            pagg_hbm.at[pl.ds(15 * RPT, RPT_LAST),
                                        pl.ds(col0, DH)])

        if with_deg:
            @pl.when(s == 0)
            def _():
                pltpu.sync_copy(deg_sh, pdeg_hbm.at[c])

    return sc_pass


_sc_pass_deg = _make_sc_pass(True)
_sc_pass = _make_sc_pass(False)

BLK = 5000


def _tc_self_body(x_ref, w_ref, b_ref, o_ref):
    o_ref[...] = jnp.dot(x_ref[...], w_ref[...],
                         preferred_element_type=jnp.float32) + b_ref[...]


def _tc_self(h, w, b):
    return pl.pallas_call(
        _tc_self_body,
        grid=(N_NODES // BLK,),
        in_specs=[
            pl.BlockSpec((BLK, D), lambda i: (i, 0)),
            pl.BlockSpec((D, D), lambda i: (0, 0)),
            pl.BlockSpec((1, D), lambda i: (0, 0)),
        ],
        out_specs=pl.BlockSpec((BLK, D), lambda i: (i, 0)),
        out_shape=jax.ShapeDtypeStruct((N_NODES, D), jnp.float32),
    )(h, w[:D], b)


def _tc_combine_body(relu, self_ref, pagg_ref, pdeg_ref, w_ref, o_ref):
    deg = jnp.maximum(jnp.sum(pdeg_ref[...], axis=1), 1.0)
    agg = pagg_ref[...] * (1.0 / deg)[:, None]
    acc = self_ref[...] + jnp.dot(agg, w_ref[...],
                                  preferred_element_type=jnp.float32)
    if relu:
        acc = jnp.maximum(acc, 0.0)
    o_ref[...] = acc


def _tc_combine(relu, self_, pagg, pdeg, w):
    return pl.pallas_call(
        functools.partial(_tc_combine_body, relu),
        grid=(N_NODES // BLK,),
        in_specs=[
            pl.BlockSpec((BLK, D), lambda i: (i, 0)),
            pl.BlockSpec((BLK, D), lambda i: (i, 0)),
            pl.BlockSpec((BLK, 2), lambda i: (i, 0)),
            pl.BlockSpec((D, D), lambda i: (0, 0)),
        ],
        out_specs=pl.BlockSpec((BLK, D), lambda i: (i, 0)),
        out_shape=jax.ShapeDtypeStruct((N_NODES, D), jnp.float32),
    )(self_, pagg, pdeg, w[D:])


def kernel(x, edge_index, W1, b1, W2, b2):
    src = edge_index[0]
    src01 = jnp.stack([2 * src, 2 * src + 1]).reshape(2, NS, CHUNKS, C)
    dst3 = edge_index[1].reshape(NS, CHUNKS, C)
    self1 = _tc_self(x, W1, b1.reshape(1, D))
    pagg1, pdeg = _sc_pass_deg(x.reshape(2 * N_NODES, DH), src01, dst3)
    pdeg2d = pdeg.T
    h1 = _tc_combine(True, self1, pagg1, pdeg2d, W1)
    self2 = _tc_self(h1, W2, b2.reshape(1, D))
    pagg2 = _sc_pass(h1.reshape(2 * N_NODES, DH), src01, dst3)
    h2 = _tc_combine(False, self2, pagg2, pdeg2d, W2)
    return h2
